# Initial kernel scaffold; baseline (speedup 1.0000x reference)
import jax
import jax.numpy as jnp
from jax.experimental import pallas as pl


def kernel(venueid2coor, inputs_poi, poi_distance_matrix):
    raise NotImplementedError("write your pallas kernel here")


# speedup vs baseline: 1.6826x; 1.6826x over previous
import functools

import jax
import jax.numpy as jnp
from jax import lax
from jax.experimental import pallas as pl
from jax.experimental.pallas import tpu as pltpu
from jax.experimental.pallas import tpu_sc as plsc

N_ROWS = 1000
B_TOTAL = 1024 * 20
NW = 32
B_PER_W = B_TOTAL // NW
CHUNK = 80
N_CHUNKS = B_PER_W // CHUNK
L = 16


def _recip_body(x_ref, o_ref):
    x = x_ref[...]
    d = jnp.where(x == 0.0, jnp.float32(9999999.99), x)
    o_ref[...] = 1.0 / d


_recip_call = pl.pallas_call(
    _recip_body,
    out_shape=jax.ShapeDtypeStruct((N_ROWS, N_ROWS), jnp.float32),
)


_sc_mesh = plsc.VectorSubcoreMesh(core_axis_name="c", subcore_axis_name="s")


@functools.partial(
    pl.kernel,
    mesh=_sc_mesh,
    out_type=jax.ShapeDtypeStruct((B_TOTAL, N_ROWS), jnp.float32),
    compiler_params=pltpu.CompilerParams(use_tc_tiling_on_sc=False),
    scratch_types=[
        pltpu.VMEM((CHUNK,), jnp.int32),
        pltpu.VMEM((CHUNK,), jnp.int32),
        pltpu.VMEM((CHUNK, N_ROWS), jnp.float32),
        pltpu.SemaphoreType.DMA,
    ],
)
def _sc_gather(venue_hbm, poi_hbm, table_hbm, out_hbm,
               poi_v, idx_v, rows_v, sem):
    wid = lax.axis_index("s") * 2 + lax.axis_index("c")
    base_w = wid * B_PER_W

    def chunk_body(j, carry):
        base = base_w + j * CHUNK
        pltpu.sync_copy(poi_hbm.at[pl.ds(base, CHUNK)], poi_v)
        pltpu.async_copy(venue_hbm.at[poi_v], idx_v, sem).wait()
        pltpu.async_copy(table_hbm.at[idx_v], rows_v, sem).wait()
        pltpu.sync_copy(rows_v, out_hbm.at[pl.ds(base, CHUNK)])
        return carry

    lax.fori_loop(0, N_CHUNKS, chunk_body, 0)


def kernel(venueid2coor, inputs_poi, poi_distance_matrix):
    recip = _recip_call(poi_distance_matrix)
    poi_flat = inputs_poi.reshape(-1)
    out = _sc_gather(venueid2coor, poi_flat, recip)
    return out.reshape(inputs_poi.shape[0], inputs_poi.shape[1], N_ROWS)
